# Initial kernel scaffold; baseline (speedup 1.0000x reference)
import jax
import jax.numpy as jnp
from jax.experimental import pallas as pl


def kernel(x, edge_index, batch, eps, W1, b1, W_mu, b_mu, W_lv, b_lv):
    raise NotImplementedError("write your pallas kernel here")


# speedup vs baseline: 3.8255x; 3.8255x over previous
import functools

import jax
import jax.numpy as jnp
from jax.experimental import pallas as pl
from jax.experimental.pallas import tpu as pltpu


def _decoder_body(z_row_ref, z_col_ref, out_ref):
    logits = jax.lax.dot_general(
        z_row_ref[...], z_col_ref[...],
        (((1,), (1,)), ((), ())),
        preferred_element_type=jnp.float32,
    )
    out_ref[...] = jax.nn.sigmoid(logits)


def _decode(z):
    n, lat = z.shape
    br = 400
    return pl.pallas_call(
        _decoder_body,
        grid=(n // br,),
        in_specs=[
            pl.BlockSpec((br, lat), lambda i: (i, 0)),
            pl.BlockSpec((n, lat), lambda i: (0, 0)),
        ],
        out_specs=pl.BlockSpec((br, n), lambda i: (i, 0)),
        out_shape=jax.ShapeDtypeStruct((n, n), jnp.float32),
    )(z, z)


def kernel(x, edge_index, batch, eps, W1, b1, W_mu, b_mu, W_lv, b_lv):
    n = x.shape[0]
    src = edge_index[0]
    dst = edge_index[1]
    deg = jnp.ones((n,), jnp.float32).at[dst].add(1.0)
    dis = jax.lax.rsqrt(jnp.maximum(deg, 1.0))

    xs = x * dis[:, None]
    agg1 = jnp.zeros_like(xs).at[dst].add(xs[src]) + xs
    h = jax.nn.relu((agg1 * dis[:, None]) @ W1 + b1)

    wcat = jnp.concatenate([W_mu, W_lv], axis=1)
    hm = (h @ wcat) * dis[:, None]
    agg2 = jnp.zeros_like(hm).at[dst].add(hm[src]) + hm
    mulv = agg2 * dis[:, None] + jnp.concatenate([b_mu, b_lv])
    mu = mulv[:, :32]
    logvar = mulv[:, 32:]

    z = mu + eps * jnp.exp(0.5 * logvar)
    adj_hat = _decode(z)
    return (adj_hat, mu, logvar)


# speedup vs baseline: 12.5051x; 3.2689x over previous
import functools

import jax
import jax.numpy as jnp
from jax import lax
from jax.experimental import pallas as pl
from jax.experimental.pallas import tpu as pltpu
from jax.experimental.pallas import tpu_sc as plsc

NP = 10240
CHUNK = 128
NCORES = 2
NSUB = 16
NWORK = NCORES * NSUB
RPT = NP // NSUB

_mesh = functools.partial(
    plsc.VectorSubcoreMesh, core_axis_name="c", subcore_axis_name="s")


def _sc_degree(dst2d, n_nodes):
    cpw = dst2d.shape[0] // NWORK

    def body(dst_hbm, out_hbm, dst_v, ones_v, lbuf, deg_sh, sem):
        c = lax.axis_index("c")
        s = lax.axis_index("s")
        w = c * NSUB + s
        for r in range(CHUNK // 16):
            ones_v[pl.ds(r * 16, 16)] = jnp.ones((16,), jnp.float32)

        def zr(i, carry):
            lbuf[pl.ds(i * 16, 16)] = jnp.zeros((16,), jnp.float32)
            return carry

        lax.fori_loop(0, RPT // 16, zr, 0)
        pltpu.sync_copy(lbuf, deg_sh.at[pl.ds(s * RPT, RPT)])
        plsc.subcore_barrier()

        pltpu.sync_copy(dst_hbm.at[pl.ds(w * cpw, cpw)], dst_v)

        def step(j, carry):
            pltpu.sync_copy(ones_v, deg_sh.at[dst_v.at[j]], add=True)
            return carry

        lax.fori_loop(0, cpw, step, 0)
        plsc.subcore_barrier()
        pltpu.sync_copy(deg_sh.at[pl.ds(s * RPT, RPT)], lbuf)
        pltpu.sync_copy(lbuf, out_hbm.at[c, pl.ds(s * RPT, RPT)])

    fn = pl.kernel(
        body,
        out_type=jax.ShapeDtypeStruct((NCORES, NP), jnp.float32),
        mesh=_mesh(),
        scratch_types=[
            pltpu.VMEM((cpw, CHUNK), jnp.int32),
            pltpu.VMEM((CHUNK,), jnp.float32),
            pltpu.VMEM((RPT,), jnp.float32),
            pltpu.VMEM_SHARED((NP,), jnp.float32),
            pltpu.SemaphoreType.DMA,
        ],
    )
    return fn(dst2d)


def _sc_segsum(feat, src2d, dst2d, width):
    nph = len(feat)
    cpw = src2d.shape[0] // NWORK

    def body(*refs):
        feat_hbms = refs[:nph]
        src_hbm, dst_hbm, out_hbm = refs[nph:nph + 3]
        src_v, dst_v, rows_a, rows_b, acc, sem_a, sem_b = refs[nph + 3:]
        c = lax.axis_index("c")
        s = lax.axis_index("s")
        w = c * NSUB + s

        pltpu.sync_copy(src_hbm.at[pl.ds(w * cpw, cpw)], src_v)
        pltpu.sync_copy(dst_hbm.at[pl.ds(w * cpw, cpw)], dst_v)

        for p, feat_hbm in enumerate(feat_hbms):
            def zr(i, carry):
                for q in range(width // 16):
                    rows_a[i, pl.ds(q * 16, 16)] = jnp.zeros((16,),
                                                             jnp.float32)
                return carry

            lax.fori_loop(0, CHUNK, zr, 0)
            for t in range(RPT // CHUNK):
                pltpu.sync_copy(
                    rows_a, acc.at[pl.ds(s * RPT + t * CHUNK, CHUNK)])
            plsc.subcore_barrier()

            pltpu.async_copy(feat_hbm.at[src_v.at[0]], rows_a, sem_a)

            def step(j, carry):
                even = lax.rem(j, 2) == 0

                def do(rows_cur, sem_cur, rows_nxt, sem_nxt):
                    pltpu.make_async_copy(feat_hbm.at[src_v.at[j]], rows_cur,
                                          sem_cur).wait()

                    @pl.when(j + 1 < cpw)
                    def _():
                        pltpu.async_copy(feat_hbm.at[src_v.at[j + 1]],
                                         rows_nxt, sem_nxt)

                    pltpu.sync_copy(rows_cur, acc.at[dst_v.at[j]], add=True)

                @pl.when(even)
                def _():
                    do(rows_a, sem_a, rows_b, sem_b)

                @pl.when(jnp.logical_not(even))
                def _():
                    do(rows_b, sem_b, rows_a, sem_a)

                return carry

            lax.fori_loop(0, cpw, step, 0)
            plsc.subcore_barrier()

            for t in range(RPT // CHUNK):
                base = s * RPT + t * CHUNK
                pltpu.sync_copy(acc.at[pl.ds(base, CHUNK)], rows_a)
                pltpu.sync_copy(rows_a, out_hbm.at[p, c, pl.ds(base, CHUNK)])
            plsc.subcore_barrier()

    fn = pl.kernel(
        body,
        out_type=jax.ShapeDtypeStruct((nph, NCORES, NP, width), jnp.float32),
        mesh=_mesh(),
        compiler_params=pltpu.CompilerParams(use_tc_tiling_on_sc=False),
        scratch_types=[
            pltpu.VMEM((cpw, CHUNK), jnp.int32),
            pltpu.VMEM((cpw, CHUNK), jnp.int32),
            pltpu.VMEM((CHUNK, width), jnp.float32),
            pltpu.VMEM((CHUNK, width), jnp.float32),
            pltpu.VMEM_SHARED((NP, width), jnp.float32),
            pltpu.SemaphoreType.DMA,
            pltpu.SemaphoreType.DMA,
        ],
    )
    return fn(*feat, src2d, dst2d)


_BR = 1280


def _prep_body(d0_ref, d1_ref, x_ref, xs0_ref, xs1_ref, dis_ref):
    deg = d0_ref[0] + d1_ref[0] + 1.0
    dis = lax.rsqrt(jnp.maximum(deg, 1.0))
    dis_ref[...] = dis
    xs = x_ref[...] * dis
    xs0_ref[...] = xs[:, :64]
    xs1_ref[...] = xs[:, 64:]


def _prep(deg_parts, x_pad):
    d2 = deg_parts.reshape(NCORES, NP, 1)
    return pl.pallas_call(
        _prep_body,
        grid=(NP // _BR,),
        in_specs=[
            pl.BlockSpec((1, _BR, 1), lambda i: (0, i, 0)),
            pl.BlockSpec((1, _BR, 1), lambda i: (1, i, 0)),
            pl.BlockSpec((_BR, 128), lambda i: (i, 0)),
        ],
        out_specs=[
            pl.BlockSpec((_BR, 64), lambda i: (i, 0)),
            pl.BlockSpec((_BR, 64), lambda i: (i, 0)),
            pl.BlockSpec((_BR, 1), lambda i: (i, 0)),
        ],
        out_shape=[
            jax.ShapeDtypeStruct((NP, 64), jnp.float32),
            jax.ShapeDtypeStruct((NP, 64), jnp.float32),
            jax.ShapeDtypeStruct((NP, 1), jnp.float32),
        ],
    )(d2, d2, x_pad)


def _mid_body(p00_ref, p01_ref, p10_ref, p11_ref, xs0_ref, xs1_ref, dis_ref,
              w1_ref, b1_ref, wc_ref, hm_ref):
    dis = dis_ref[...]
    agg_a = (p00_ref[0, 0] + p01_ref[0, 0] + xs0_ref[...]) * dis
    agg_b = (p10_ref[0, 0] + p11_ref[0, 0] + xs1_ref[...]) * dis
    w1 = w1_ref[...]
    pre = (jnp.dot(agg_a, w1[:64, :], precision=lax.Precision.HIGHEST,
                   preferred_element_type=jnp.float32)
           + jnp.dot(agg_b, w1[64:, :], precision=lax.Precision.HIGHEST,
                     preferred_element_type=jnp.float32) + b1_ref[...])
    h = jnp.maximum(pre, 0.0)
    hm_ref[...] = jnp.dot(h, wc_ref[...], precision=lax.Precision.HIGHEST,
                          preferred_element_type=jnp.float32) * dis


def _mid(parts, xs0, xs1, dis, W1, b1, Wcat):
    pspec = lambda p, c: pl.BlockSpec(
        (1, 1, _BR, 64), lambda i, p=p, c=c: (p, c, i, 0))
    return pl.pallas_call(
        _mid_body,
        grid=(NP // _BR,),
        in_specs=[
            pspec(0, 0), pspec(0, 1), pspec(1, 0), pspec(1, 1),
            pl.BlockSpec((_BR, 64), lambda i: (i, 0)),
            pl.BlockSpec((_BR, 64), lambda i: (i, 0)),
            pl.BlockSpec((_BR, 1), lambda i: (i, 0)),
            pl.BlockSpec((128, 256), lambda i: (0, 0)),
            pl.BlockSpec((1, 256), lambda i: (0, 0)),
            pl.BlockSpec((256, 64), lambda i: (0, 0)),
        ],
        out_specs=pl.BlockSpec((_BR, 64), lambda i: (i, 0)),
        out_shape=jax.ShapeDtypeStruct((NP, 64), jnp.float32),
    )(parts, parts, parts, parts, xs0, xs1, dis, W1, b1.reshape(1, 256),
      Wcat)


def _z_body(q0_ref, q1_ref, hm_ref, dis_ref, bc_ref, eps_ref,
            mu_ref, lv_ref, z_ref):
    mulv = ((q0_ref[0] + q1_ref[0] + hm_ref[...]) * dis_ref[...]
            + bc_ref[...])
    mu = mulv[:, :32]
    lv = mulv[:, 32:]
    mu_ref[...] = mu
    lv_ref[...] = lv
    z_ref[...] = mu + eps_ref[...] * jnp.exp(0.5 * lv)


def _zstage(qparts, hm, dis, bcat, eps_pad):
    q3 = qparts.reshape(NCORES, NP, 64)
    return pl.pallas_call(
        _z_body,
        grid=(NP // _BR,),
        in_specs=[
            pl.BlockSpec((1, _BR, 64), lambda i: (0, i, 0)),
            pl.BlockSpec((1, _BR, 64), lambda i: (1, i, 0)),
            pl.BlockSpec((_BR, 64), lambda i: (i, 0)),
            pl.BlockSpec((_BR, 1), lambda i: (i, 0)),
            pl.BlockSpec((1, 64), lambda i: (0, 0)),
            pl.BlockSpec((_BR, 32), lambda i: (i, 0)),
        ],
        out_specs=[
            pl.BlockSpec((_BR, 32), lambda i: (i, 0)),
            pl.BlockSpec((_BR, 32), lambda i: (i, 0)),
            pl.BlockSpec((_BR, 32), lambda i: (i, 0)),
        ],
        out_shape=[
            jax.ShapeDtypeStruct((NP, 32), jnp.float32),
            jax.ShapeDtypeStruct((NP, 32), jnp.float32),
            jax.ShapeDtypeStruct((NP, 32), jnp.float32),
        ],
    )(q3, q3, hm, dis, bcat.reshape(1, 64), eps_pad)


def _decoder_body(z_row_ref, z_col_ref, out_ref):
    logits = lax.dot_general(
        z_row_ref[...], z_col_ref[...],
        (((1,), (1,)), ((), ())),
        preferred_element_type=jnp.float32,
    )
    out_ref[...] = jax.nn.sigmoid(logits)


def _decode(z):
    n, latd = z.shape
    br = 400
    return pl.pallas_call(
        _decoder_body,
        grid=(n // br,),
        in_specs=[
            pl.BlockSpec((br, latd), lambda i: (i, 0)),
            pl.BlockSpec((n, latd), lambda i: (0, 0)),
        ],
        out_specs=pl.BlockSpec((br, n), lambda i: (i, 0)),
        out_shape=jax.ShapeDtypeStruct((n, n), jnp.float32),
    )(z, z)


def kernel(x, edge_index, batch, eps, W1, b1, W_mu, b_mu, W_lv, b_lv):
    n = x.shape[0]
    e = edge_index.shape[1]
    src = edge_index[0].astype(jnp.int32)
    dst = edge_index[1].astype(jnp.int32)

    cpw = -(-e // (CHUNK * NWORK))
    cpw = -(-cpw // 8) * 8
    ep = CHUNK * NWORK * cpw
    src_p = jnp.concatenate(
        [src, jnp.zeros((ep - e,), jnp.int32)]).reshape(NWORK * cpw, CHUNK)
    dst_p = jnp.concatenate(
        [dst, jnp.full((ep - e,), n, jnp.int32)]).reshape(NWORK * cpw, CHUNK)

    x_pad = jnp.pad(x, ((0, NP - n), (0, 0)))
    eps_pad = jnp.pad(eps, ((0, NP - n), (0, 0)))

    deg_parts = _sc_degree(dst_p, n)
    xs0, xs1, dis = _prep(deg_parts, x_pad)
    parts1 = _sc_segsum([xs0, xs1], src_p, dst_p, 64)
    hm = _mid(parts1, xs0, xs1, dis, W1, b1,
              jnp.concatenate([W_mu, W_lv], axis=1))
    parts2 = _sc_segsum([hm], src_p, dst_p, 64)
    mu_p, lv_p, z_p = _zstage(parts2, hm, dis,
                              jnp.concatenate([b_mu, b_lv]), eps_pad)
    adj_hat = _decode(z_p[:n])
    return (adj_hat, mu_p[:n], lv_p[:n])
